# baseline (speedup 1.0000x reference)
import functools

import jax
import jax.numpy as jnp
from jax import lax
from jax.experimental import pallas as pl
from jax.experimental.pallas import tpu as pltpu
from jax.experimental.pallas import tpu_sc as plsc

N = 10000
E = 160000
D = 256
H = 512
G = 128
NCLS = 10

CH = 128
NCHUNK = H // CH

NSUB = 16
K = 40
EPS = E // NSUB
NBLK = EPS // K
NPAD = 10240
ROWS = NPAD // NSUB
ZR = 40
BN_ROWS = 1000
F32 = jnp.float32


def _sc_aggregate(h4, ea4f, src, dst):
    mesh = plsc.VectorSubcoreMesh(core_axis_name="c", subcore_axis_name="s")
    vg = CH // 16

    @functools.partial(
        pl.kernel,
        mesh=mesh,
        out_type=jax.ShapeDtypeStruct((NCHUNK * N, CH), F32),
        scratch_types=[
            pltpu.VMEM_SHARED((NPAD, CH), F32),
            pltpu.VMEM((K,), jnp.int32),
            pltpu.VMEM((K,), jnp.int32),
            pltpu.VMEM((K,), jnp.int32),
            pltpu.VMEM((K,), jnp.int32),
            pltpu.VMEM((K,), jnp.int32),
            pltpu.VMEM((K,), jnp.int32),
            pltpu.VMEM((K,), jnp.int32),
            pltpu.VMEM((K,), jnp.int32),
            pltpu.VMEM((K, CH), F32),
            pltpu.VMEM((K, CH), F32),
            pltpu.VMEM((K, CH), F32),
            pltpu.VMEM((K * CH // 2,), jnp.int32),
            pltpu.VMEM((K * CH // 2,), jnp.int32),
            pltpu.SemaphoreType.DMA,
            pltpu.SemaphoreType.DMA,
            pltpu.SemaphoreType.DMA,
            pltpu.SemaphoreType.DMA,
            pltpu.SemaphoreType.DMA,
            pltpu.SemaphoreType.DMA,
            pltpu.SemaphoreType.DMA,
            pltpu.SemaphoreType.DMA,
            pltpu.SemaphoreType.DMA,
            pltpu.SemaphoreType.DMA,
            pltpu.SemaphoreType.DMA,
            pltpu.SemaphoreType.DMA,
            pltpu.SemaphoreType.DMA,
            pltpu.SemaphoreType.DMA,
        ],
    )
    def agg_kernel(h_hbm, ea_hbm, src_hbm, dst_hbm, out_hbm,
                   table, s0, s1, s2, s3, d0, d1, d2, d3,
                   h0, h1, h2, e0, e1,
                   sg0, sg1, sg2, se0, se1, semsc,
                   sr0, sr1, sr2, sr3, sd0, sd1, sd2, sd3):
        core = lax.axis_index("c")
        sub = lax.axis_index("s")
        hrows = (h0, h1, h2)
        earows = (e0, e1)
        srcv = (s0, s1, s2, s3)
        dstv = (d0, d1, d2, d3)
        semg = (sg0, sg1, sg2)
        seme = (se0, se1)
        semr = (sr0, sr1, sr2, sr3)
        semd = (sd0, sd1, sd2, sd3)

        zeros16 = jnp.zeros((16,), F32)
        row0 = sub * ROWS

        for cc in range(NCHUNK // 2):
            c = core * (NCHUNK // 2) + cc
            ebase = c * E + sub * EPS
            obase = c * N
            hview = h_hbm.at[c]

            def zb(r, cy):
                for j in range(vg):
                    h0[r, pl.ds(j * 16, 16)] = zeros16
                return cy

            lax.fori_loop(0, ZR, zb, 0)
            for z in range(ROWS // ZR):
                pltpu.sync_copy(h0, table.at[pl.ds(row0 + z * ZR, ZR)])
            plsc.subcore_barrier()

            def issue_idx(b, t):
                pi = t % 4
                pltpu.async_copy(src_hbm.at[pl.ds(sub * EPS + b * K, K)],
                                 srcv[pi], semr[pi])
                pltpu.async_copy(dst_hbm.at[pl.ds(sub * EPS + b * K, K)],
                                 dstv[pi], semd[pi])

            def issue_rows(b, t):
                pg, pe, pi = t % 3, t % 2, t % 4
                pltpu.make_async_copy(
                    src_hbm.at[pl.ds(sub * EPS + b * K, K)], srcv[pi],
                    semr[pi]).wait()
                pltpu.async_copy(hview.at[srcv[pi]], hrows[pg], semg[pg])
                pltpu.async_copy(
                    ea_hbm.at[pl.ds((ebase + b * K) * (CH // 2), K * CH // 2)],
                    earows[pe], seme[pe])

            def drain_scatter(pg, pi):
                pltpu.make_async_copy(hrows[pg], table.at[dstv[pi]],
                                      semsc).wait()

            def step(b, t):
                pg, pe, pi = t % 3, t % 2, t % 4

                @pl.when(b + 2 < NBLK)
                def _():
                    issue_idx(b + 2, t + 2)

                @pl.when(b + 1 < NBLK)
                def _():
                    issue_rows(b + 1, t + 1)

                @pl.when(b < NBLK)
                def _():
                    pltpu.make_async_copy(
                        hview.at[srcv[pi]], hrows[pg], semg[pg]).wait()
                    pltpu.make_async_copy(
                        ea_hbm.at[pl.ds((ebase + b * K) * (CH // 2),
                                        K * CH // 2)],
                        earows[pe], seme[pe]).wait()

                    def rel(r, cy):
                        rbase = pl.multiple_of(r * (CH // 2), CH // 2)
                        for j in range(CH // 32):
                            w = earows[pe][pl.ds(rbase + j * 16, 16)]
                            lo = lax.bitcast_convert_type(w << 16, F32)
                            hi = lax.bitcast_convert_type(
                                w & jnp.int32(-65536), F32)
                            sl0 = pl.ds(j * 32, 16)
                            sl1 = pl.ds(j * 32 + 16, 16)
                            v0 = hrows[pg][r, sl0] + lo
                            v1 = hrows[pg][r, sl1] + hi
                            hrows[pg][r, sl0] = jnp.maximum(v0, 0.0)
                            hrows[pg][r, sl1] = jnp.maximum(v1, 0.0)
                        return cy

                    lax.fori_loop(0, K, rel, 0)
                    pltpu.make_async_copy(
                        dst_hbm.at[pl.ds(sub * EPS + b * K, K)], dstv[pi],
                        semd[pi]).wait()

                    @pl.when(b >= 1)
                    def _():
                        drain_scatter((t + 2) % 3, (t + 3) % 4)

                    pltpu.async_copy(hrows[pg], table.at[dstv[pi]],
                                     semsc, add=True)

            issue_idx(0, 0)
            issue_idx(1, 1)
            issue_rows(0, 0)

            def grp(g, cy):
                for t in range(12):
                    step(12 * g + t, t)
                return cy

            lax.fori_loop(0, (NBLK + 12 - 1) // 12, grp, 0)
            drain_scatter((NBLK - 1) % 3, (NBLK - 1) % 4)
            plsc.subcore_barrier()

            for z in range(ROWS // ZR):
                off = row0 + z * ZR

                @pl.when(off + ZR <= N)
                def _():
                    pltpu.sync_copy(table.at[pl.ds(off, ZR)], h0)
                    pltpu.sync_copy(h0, out_hbm.at[pl.ds(obase + off, ZR)])

            plsc.subcore_barrier()

    return agg_kernel(h4, ea4f, src, dst)


def _encode_body(x_ref, w_ref, b_ref, o_ref):
    xb = x_ref[...]
    for c in range(NCHUNK):
        z = jnp.dot(xb, w_ref[c], preferred_element_type=F32) + b_ref[c]
        o_ref[c] = z.astype(o_ref.dtype)


def _encode(x, wr, br, rows, block_rows, dtype=F32):
    din = x.shape[1]
    grid = (rows // block_rows,)
    return pl.pallas_call(
        _encode_body,
        grid=grid,
        in_specs=[
            pl.BlockSpec((block_rows, din), lambda i: (i, 0)),
            pl.BlockSpec((NCHUNK, din, CH), lambda i: (0, 0, 0)),
            pl.BlockSpec((NCHUNK, 1, CH), lambda i: (0, 0, 0)),
        ],
        out_specs=pl.BlockSpec((NCHUNK, block_rows, CH), lambda i: (0, i, 0)),
        out_shape=jax.ShapeDtypeStruct((NCHUNK, rows, CH), dtype),
    )(x, wr, br)


def _mlp1_body(h_ref, a_ref, w_ref, b_ref, z_ref, s_ref):
    acc = jnp.zeros((BN_ROWS, H), F32)
    for c in range(NCHUNK):
        acc += jnp.dot(h_ref[c] + a_ref[c], w_ref[c],
                       preferred_element_type=F32)
    z = acc + b_ref[...]
    z_ref[...] = z
    delta = jnp.concatenate(
        [jnp.sum(z, axis=0, keepdims=True),
         jnp.sum(z * z, axis=0, keepdims=True)], axis=0)

    @pl.when(pl.program_id(0) == 0)
    def _():
        s_ref[...] = delta

    @pl.when(pl.program_id(0) > 0)
    def _():
        s_ref[...] += delta


def _mlp1(h4, agg4, w1r, b1r):
    grid = (N // BN_ROWS,)
    return pl.pallas_call(
        _mlp1_body,
        grid=grid,
        in_specs=[
            pl.BlockSpec((NCHUNK, BN_ROWS, CH), lambda i: (0, i, 0)),
            pl.BlockSpec((NCHUNK, BN_ROWS, CH), lambda i: (0, i, 0)),
            pl.BlockSpec((NCHUNK, CH, H), lambda i: (0, 0, 0)),
            pl.BlockSpec((1, H), lambda i: (0, 0)),
        ],
        out_specs=[
            pl.BlockSpec((BN_ROWS, H), lambda i: (i, 0)),
            pl.BlockSpec((2, H), lambda i: (0, 0)),
        ],
        out_shape=[
            jax.ShapeDtypeStruct((N, H), F32),
            jax.ShapeDtypeStruct((2, H), F32),
        ],
    )(h4, agg4, w1r, b1r)


def _mlp2_body(z_ref, s_ref, g_ref, bb_ref, w_ref, b_ref, o_ref):
    mu = s_ref[0:1, :] * (1.0 / N)
    var = s_ref[1:2, :] * (1.0 / N) - mu * mu
    scale = g_ref[...] * lax.rsqrt(var + 1e-5)
    shift = bb_ref[...] - mu * scale
    zn = jnp.maximum(z_ref[...] * scale + shift, 0.0)
    for c in range(NCHUNK):
        o_ref[c] = jnp.maximum(
            jnp.dot(zn, w_ref[c], preferred_element_type=F32) + b_ref[c], 0.0)


def _mlp2(z1, stats, gr, bbr, w2r, b2r):
    grid = (N // BN_ROWS,)
    return pl.pallas_call(
        _mlp2_body,
        grid=grid,
        in_specs=[
            pl.BlockSpec((BN_ROWS, H), lambda i: (i, 0)),
            pl.BlockSpec((2, H), lambda i: (0, 0)),
            pl.BlockSpec((1, H), lambda i: (0, 0)),
            pl.BlockSpec((1, H), lambda i: (0, 0)),
            pl.BlockSpec((NCHUNK, H, CH), lambda i: (0, 0, 0)),
            pl.BlockSpec((NCHUNK, 1, CH), lambda i: (0, 0, 0)),
        ],
        out_specs=pl.BlockSpec((NCHUNK, BN_ROWS, CH), lambda i: (0, i, 0)),
        out_shape=jax.ShapeDtypeStruct((NCHUNK, N, CH), F32),
    )(z1, stats, gr, bbr, w2r, b2r)


def _pool_body(h_ref, batch_ref, wo_ref, bo_ref, o_ref):
    b = batch_ref[0, 0, :]
    oh = (b[:, None] == lax.broadcasted_iota(jnp.int32, (BN_ROWS, G), 1)
          ).astype(F32)
    hb = jnp.concatenate([h_ref[c] for c in range(NCHUNK)], axis=1)
    pooled = lax.dot_general(oh, hb, (((0,), (0,)), ((), ())),
                             preferred_element_type=F32)
    contrib = jnp.dot(pooled, wo_ref[...], preferred_element_type=F32)

    @pl.when(pl.program_id(0) == 0)
    def _():
        o_ref[...] = contrib + bo_ref[...]

    @pl.when(pl.program_id(0) > 0)
    def _():
        o_ref[...] += contrib


def _pool(h4, batch3, wout, bout2):
    grid = (N // BN_ROWS,)
    nb = N // BN_ROWS
    return pl.pallas_call(
        _pool_body,
        grid=grid,
        in_specs=[
            pl.BlockSpec((NCHUNK, BN_ROWS, CH), lambda i: (0, i, 0)),
            pl.BlockSpec((1, 1, BN_ROWS), lambda i: (i, 0, 0)),
            pl.BlockSpec((H, NCLS), lambda i: (0, 0)),
            pl.BlockSpec((1, NCLS), lambda i: (0, 0)),
        ],
        out_specs=pl.BlockSpec((G, NCLS), lambda i: (0, 0)),
        out_shape=jax.ShapeDtypeStruct((G, NCLS), F32),
    )(h4, batch3, wout, bout2)


def kernel(x, edge_index, batch, edge_attr, Wn, bnv, We, bev,
           L0_W1, L0_b1, L0_g, L0_bb, L0_W2, L0_b2,
           L1_W1, L1_b1, L1_g, L1_bb, L1_W2, L1_b2,
           L2_W1, L2_b1, L2_g, L2_bb, L2_W2, L2_b2,
           Wout, bout):
    src = edge_index[0]
    dst = edge_index[1]

    wnr = Wn.reshape(D, NCHUNK, CH).transpose(1, 0, 2)
    bnr = bnv.reshape(NCHUNK, 1, CH)

    idx = jnp.arange(H)
    gg, rr = idx // 32, idx % 32
    perm = jnp.where(rr % 2 == 0, 32 * gg + rr // 2, 32 * gg + 16 + rr // 2)
    wep = We[:, perm]
    bep = bev[perm]
    wer = wep.reshape(edge_attr.shape[1], NCHUNK, CH).transpose(1, 0, 2)
    ber = bep.reshape(NCHUNK, 1, CH)

    h4 = _encode(x, wnr, bnr, N, BN_ROWS)
    ea4 = _encode(edge_attr, wer, ber, E, 2000, dtype=jnp.bfloat16)
    ea4f = lax.bitcast_convert_type(
        ea4.reshape(NCHUNK, E, CH // 2, 2), jnp.int32
    ).reshape(NCHUNK * E * (CH // 2))

    layers = [
        (L0_W1, L0_b1, L0_g, L0_bb, L0_W2, L0_b2),
        (L1_W1, L1_b1, L1_g, L1_bb, L1_W2, L1_b2),
        (L2_W1, L2_b1, L2_g, L2_bb, L2_W2, L2_b2),
    ]
    for (W1, b1, g, bb, W2, b2) in layers:
        aggf = _sc_aggregate(h4, ea4f, src, dst)
        agg4 = aggf.reshape(NCHUNK, N, CH)
        w1r = W1.reshape(NCHUNK, CH, H)
        w2r = W2.reshape(H, NCHUNK, CH).transpose(1, 0, 2)
        z1, stats = _mlp1(h4, agg4, w1r, b1.reshape(1, H))
        h4 = _mlp2(z1, stats, g.reshape(1, H), bb.reshape(1, H),
                   w2r, b2.reshape(NCHUNK, 1, CH))

    batch3 = batch.reshape(N // BN_ROWS, 1, BN_ROWS)
    return _pool(h4, batch3, Wout, bout.reshape(1, NCLS))


# speedup vs baseline: 1.3655x; 1.3655x over previous
import functools

import jax
import jax.numpy as jnp
from jax import lax
from jax.experimental import pallas as pl
from jax.experimental.pallas import tpu as pltpu
from jax.experimental.pallas import tpu_sc as plsc

N = 10000
E = 160000
D = 256
H = 512
G = 128
NCLS = 10

CH = 128
NCHUNK = H // CH

NSUB = 16
K = 80
EPS = E // NSUB
NBLK = EPS // K
NPAD = 10240
ROWS = NPAD // NSUB
ZR = 80
BN_ROWS = 1000
F32 = jnp.float32


def _sc_aggregate(h4, ea4f, src, dst):
    mesh = plsc.VectorSubcoreMesh(core_axis_name="c", subcore_axis_name="s")
    vg = CH // 16

    @functools.partial(
        pl.kernel,
        mesh=mesh,
        out_type=jax.ShapeDtypeStruct((NCHUNK * N, CH), F32),
        scratch_types=[
            pltpu.VMEM_SHARED((NPAD, CH), F32),
            pltpu.VMEM((K,), jnp.int32),
            pltpu.VMEM((K,), jnp.int32),
            pltpu.VMEM((K,), jnp.int32),
            pltpu.VMEM((K,), jnp.int32),
            pltpu.VMEM((K,), jnp.int32),
            pltpu.VMEM((K,), jnp.int32),
            pltpu.VMEM((K,), jnp.int32),
            pltpu.VMEM((K,), jnp.int32),
            pltpu.VMEM((K, CH), F32),
            pltpu.VMEM((K, CH), F32),
            pltpu.VMEM((K, CH), F32),
            pltpu.VMEM((K // 2, CH), jnp.int32),
            pltpu.VMEM((K // 2, CH), jnp.int32),
            pltpu.SemaphoreType.DMA,
            pltpu.SemaphoreType.DMA,
            pltpu.SemaphoreType.DMA,
            pltpu.SemaphoreType.DMA,
            pltpu.SemaphoreType.DMA,
            pltpu.SemaphoreType.DMA,
            pltpu.SemaphoreType.DMA,
            pltpu.SemaphoreType.DMA,
            pltpu.SemaphoreType.DMA,
            pltpu.SemaphoreType.DMA,
            pltpu.SemaphoreType.DMA,
            pltpu.SemaphoreType.DMA,
            pltpu.SemaphoreType.DMA,
            pltpu.SemaphoreType.DMA,
        ],
    )
    def agg_kernel(h_hbm, ea_hbm, src_hbm, dst_hbm, out_hbm,
                   table, s0, s1, s2, s3, d0, d1, d2, d3,
                   h0, h1, h2, e0, e1,
                   sg0, sg1, sg2, se0, se1, semsc,
                   sr0, sr1, sr2, sr3, sd0, sd1, sd2, sd3):
        core = lax.axis_index("c")
        sub = lax.axis_index("s")
        hrows = (h0, h1, h2)
        earows = (e0, e1)
        srcv = (s0, s1, s2, s3)
        dstv = (d0, d1, d2, d3)
        semg = (sg0, sg1, sg2)
        seme = (se0, se1)
        semr = (sr0, sr1, sr2, sr3)
        semd = (sd0, sd1, sd2, sd3)

        zeros16 = jnp.zeros((16,), F32)
        row0 = sub * ROWS

        for cc in range(NCHUNK // 2):
            c = core * (NCHUNK // 2) + cc
            ebase = c * E + sub * EPS
            obase = c * N
            hview = h_hbm.at[c]

            def zb(r, cy):
                for j in range(vg):
                    h0[r, pl.ds(j * 16, 16)] = zeros16
                return cy

            lax.fori_loop(0, ZR, zb, 0)
            for z in range(ROWS // ZR):
                pltpu.sync_copy(h0, table.at[pl.ds(row0 + z * ZR, ZR)])
            plsc.subcore_barrier()

            def issue_idx(b, t):
                pi = t % 4
                pltpu.async_copy(src_hbm.at[pl.ds(sub * EPS + b * K, K)],
                                 srcv[pi], semr[pi])
                pltpu.async_copy(dst_hbm.at[pl.ds(sub * EPS + b * K, K)],
                                 dstv[pi], semd[pi])

            def issue_rows(b, t):
                pg, pe, pi = t % 3, t % 2, t % 4
                pltpu.make_async_copy(
                    src_hbm.at[pl.ds(sub * EPS + b * K, K)], srcv[pi],
                    semr[pi]).wait()
                pltpu.async_copy(hview.at[srcv[pi]], hrows[pg], semg[pg])
                eoff = pl.multiple_of((ebase + b * K) // 2, 8)
                pltpu.async_copy(ea_hbm.at[pl.ds(eoff, K // 2)],
                                 earows[pe], seme[pe])

            def drain_scatter(pg, pi):
                pltpu.make_async_copy(hrows[pg], table.at[dstv[pi]],
                                      semsc).wait()

            def step(b, t):
                pg, pe, pi = t % 3, t % 2, t % 4

                @pl.when(b + 2 < NBLK)
                def _():
                    issue_idx(b + 2, t + 2)

                @pl.when(b + 1 < NBLK)
                def _():
                    issue_rows(b + 1, t + 1)

                @pl.when(b < NBLK)
                def _():
                    pltpu.make_async_copy(
                        hview.at[srcv[pi]], hrows[pg], semg[pg]).wait()
                    eoff = pl.multiple_of((ebase + b * K) // 2, 8)
                    pltpu.make_async_copy(
                        ea_hbm.at[pl.ds(eoff, K // 2)],
                        earows[pe], seme[pe]).wait()

                    def rel(r2, cy):
                        for s in range(2):
                            row = 2 * r2 + s
                            for j in range(vg // 2):
                                w = earows[pe][r2, pl.ds(s * 64 + j * 16, 16)]
                                lo = lax.bitcast_convert_type(w << 16, F32)
                                hi = lax.bitcast_convert_type(
                                    w & jnp.int32(-65536), F32)
                                sl0 = pl.ds(j * 32, 16)
                                sl1 = pl.ds(j * 32 + 16, 16)
                                v0 = hrows[pg][row, sl0] + lo
                                v1 = hrows[pg][row, sl1] + hi
                                hrows[pg][row, sl0] = jnp.maximum(v0, 0.0)
                                hrows[pg][row, sl1] = jnp.maximum(v1, 0.0)
                        return cy

                    lax.fori_loop(0, K // 2, rel, 0)
                    pltpu.make_async_copy(
                        dst_hbm.at[pl.ds(sub * EPS + b * K, K)], dstv[pi],
                        semd[pi]).wait()

                    @pl.when(b >= 1)
                    def _():
                        drain_scatter((t + 2) % 3, (t + 3) % 4)

                    pltpu.async_copy(hrows[pg], table.at[dstv[pi]],
                                     semsc, add=True)

            issue_idx(0, 0)
            issue_idx(1, 1)
            issue_rows(0, 0)

            def grp(g, cy):
                for t in range(12):
                    step(12 * g + t, t)
                return cy

            lax.fori_loop(0, (NBLK + 12 - 1) // 12, grp, 0)
            drain_scatter((NBLK - 1) % 3, (NBLK - 1) % 4)
            plsc.subcore_barrier()

            for z in range(ROWS // ZR):
                off = row0 + z * ZR

                @pl.when(off + ZR <= N)
                def _():
                    pltpu.sync_copy(table.at[pl.ds(off, ZR)], h0)
                    pltpu.sync_copy(h0, out_hbm.at[pl.ds(obase + off, ZR)])

            plsc.subcore_barrier()

    return agg_kernel(h4, ea4f, src, dst)


def _encode_body(x_ref, w_ref, b_ref, o_ref):
    xb = x_ref[...]
    for c in range(NCHUNK):
        z = jnp.dot(xb, w_ref[c], preferred_element_type=F32) + b_ref[c]
        o_ref[c] = z.astype(o_ref.dtype)


def _encode(x, wr, br, rows, block_rows, dtype=F32):
    din = x.shape[1]
    grid = (rows // block_rows,)
    return pl.pallas_call(
        _encode_body,
        grid=grid,
        in_specs=[
            pl.BlockSpec((block_rows, din), lambda i: (i, 0)),
            pl.BlockSpec((NCHUNK, din, CH), lambda i: (0, 0, 0)),
            pl.BlockSpec((NCHUNK, 1, CH), lambda i: (0, 0, 0)),
        ],
        out_specs=pl.BlockSpec((NCHUNK, block_rows, CH), lambda i: (0, i, 0)),
        out_shape=jax.ShapeDtypeStruct((NCHUNK, rows, CH), dtype),
    )(x, wr, br)


def _epack_body(x_ref, w_ref, b_ref, o_ref):
    xb = x_ref[...]
    for c in range(NCHUNK):
        z = jnp.dot(xb, w_ref[c], preferred_element_type=F32) + b_ref[c]
        zi = lax.bitcast_convert_type(
            z.astype(jnp.bfloat16).astype(F32), jnp.int32)
        a = jnp.concatenate(
            [zi[:, 128 * s + 32 * j:128 * s + 32 * j + 16]
             for s in range(2) for j in range(4)], axis=1)
        b = jnp.concatenate(
            [zi[:, 128 * s + 32 * j + 16:128 * s + 32 * j + 32]
             for s in range(2) for j in range(4)], axis=1)
        o_ref[c] = b | lax.shift_right_logical(a, 16)


def _encode_pack(xp, wd, bd, rows2, block_rows2):
    grid = (rows2 // block_rows2,)
    din = xp.shape[1]
    return pl.pallas_call(
        _epack_body,
        grid=grid,
        in_specs=[
            pl.BlockSpec((block_rows2, din), lambda i: (i, 0)),
            pl.BlockSpec((NCHUNK, din, 2 * CH), lambda i: (0, 0, 0)),
            pl.BlockSpec((NCHUNK, 1, 2 * CH), lambda i: (0, 0, 0)),
        ],
        out_specs=pl.BlockSpec((NCHUNK, block_rows2, CH),
                               lambda i: (0, i, 0)),
        out_shape=jax.ShapeDtypeStruct((NCHUNK, rows2, CH), jnp.int32),
    )(xp, wd, bd)


def _mlp1_body(h_ref, a_ref, w_ref, b_ref, z_ref, s_ref):
    acc = jnp.zeros((BN_ROWS, H), F32)
    for c in range(NCHUNK):
        acc += jnp.dot(h_ref[c] + a_ref[c], w_ref[c],
                       preferred_element_type=F32)
    z = acc + b_ref[...]
    z_ref[...] = z
    delta = jnp.concatenate(
        [jnp.sum(z, axis=0, keepdims=True),
         jnp.sum(z * z, axis=0, keepdims=True)], axis=0)

    @pl.when(pl.program_id(0) == 0)
    def _():
        s_ref[...] = delta

    @pl.when(pl.program_id(0) > 0)
    def _():
        s_ref[...] += delta


def _mlp1(h4, agg4, w1r, b1r):
    grid = (N // BN_ROWS,)
    return pl.pallas_call(
        _mlp1_body,
        grid=grid,
        in_specs=[
            pl.BlockSpec((NCHUNK, BN_ROWS, CH), lambda i: (0, i, 0)),
            pl.BlockSpec((NCHUNK, BN_ROWS, CH), lambda i: (0, i, 0)),
            pl.BlockSpec((NCHUNK, CH, H), lambda i: (0, 0, 0)),
            pl.BlockSpec((1, H), lambda i: (0, 0)),
        ],
        out_specs=[
            pl.BlockSpec((BN_ROWS, H), lambda i: (i, 0)),
            pl.BlockSpec((2, H), lambda i: (0, 0)),
        ],
        out_shape=[
            jax.ShapeDtypeStruct((N, H), F32),
            jax.ShapeDtypeStruct((2, H), F32),
        ],
    )(h4, agg4, w1r, b1r)


def _mlp2_body(z_ref, s_ref, g_ref, bb_ref, w_ref, b_ref, o_ref):
    mu = s_ref[0:1, :] * (1.0 / N)
    var = s_ref[1:2, :] * (1.0 / N) - mu * mu
    scale = g_ref[...] * lax.rsqrt(var + 1e-5)
    shift = bb_ref[...] - mu * scale
    zn = jnp.maximum(z_ref[...] * scale + shift, 0.0)
    for c in range(NCHUNK):
        o_ref[c] = jnp.maximum(
            jnp.dot(zn, w_ref[c], preferred_element_type=F32) + b_ref[c], 0.0)


def _mlp2(z1, stats, gr, bbr, w2r, b2r):
    grid = (N // BN_ROWS,)
    return pl.pallas_call(
        _mlp2_body,
        grid=grid,
        in_specs=[
            pl.BlockSpec((BN_ROWS, H), lambda i: (i, 0)),
            pl.BlockSpec((2, H), lambda i: (0, 0)),
            pl.BlockSpec((1, H), lambda i: (0, 0)),
            pl.BlockSpec((1, H), lambda i: (0, 0)),
            pl.BlockSpec((NCHUNK, H, CH), lambda i: (0, 0, 0)),
            pl.BlockSpec((NCHUNK, 1, CH), lambda i: (0, 0, 0)),
        ],
        out_specs=pl.BlockSpec((NCHUNK, BN_ROWS, CH), lambda i: (0, i, 0)),
        out_shape=jax.ShapeDtypeStruct((NCHUNK, N, CH), F32),
    )(z1, stats, gr, bbr, w2r, b2r)


def _pool_body(h_ref, batch_ref, wo_ref, bo_ref, o_ref):
    b = batch_ref[0, 0, :]
    oh = (b[:, None] == lax.broadcasted_iota(jnp.int32, (BN_ROWS, G), 1)
          ).astype(F32)
    hb = jnp.concatenate([h_ref[c] for c in range(NCHUNK)], axis=1)
    pooled = lax.dot_general(oh, hb, (((0,), (0,)), ((), ())),
                             preferred_element_type=F32)
    contrib = jnp.dot(pooled, wo_ref[...], preferred_element_type=F32)

    @pl.when(pl.program_id(0) == 0)
    def _():
        o_ref[...] = contrib + bo_ref[...]

    @pl.when(pl.program_id(0) > 0)
    def _():
        o_ref[...] += contrib


def _pool(h4, batch3, wout, bout2):
    grid = (N // BN_ROWS,)
    nb = N // BN_ROWS
    return pl.pallas_call(
        _pool_body,
        grid=grid,
        in_specs=[
            pl.BlockSpec((NCHUNK, BN_ROWS, CH), lambda i: (0, i, 0)),
            pl.BlockSpec((1, 1, BN_ROWS), lambda i: (i, 0, 0)),
            pl.BlockSpec((H, NCLS), lambda i: (0, 0)),
            pl.BlockSpec((1, NCLS), lambda i: (0, 0)),
        ],
        out_specs=pl.BlockSpec((G, NCLS), lambda i: (0, 0)),
        out_shape=jax.ShapeDtypeStruct((G, NCLS), F32),
    )(h4, batch3, wout, bout2)


def kernel(x, edge_index, batch, edge_attr, Wn, bnv, We, bev,
           L0_W1, L0_b1, L0_g, L0_bb, L0_W2, L0_b2,
           L1_W1, L1_b1, L1_g, L1_bb, L1_W2, L1_b2,
           L2_W1, L2_b1, L2_g, L2_bb, L2_W2, L2_b2,
           Wout, bout):
    src = edge_index[0]
    dst = edge_index[1]

    wnr = Wn.reshape(D, NCHUNK, CH).transpose(1, 0, 2)
    bnr = bnv.reshape(NCHUNK, 1, CH)

    de = edge_attr.shape[1]
    wec = We.reshape(de, NCHUNK, CH).transpose(1, 0, 2)
    zblk = jnp.zeros_like(wec)
    wd = jnp.concatenate(
        [jnp.concatenate([wec, zblk], axis=2),
         jnp.concatenate([zblk, wec], axis=2)], axis=1)
    bec = bev.reshape(NCHUNK, 1, CH)
    bd = jnp.concatenate([bec, bec], axis=2)

    h4 = _encode(x, wnr, bnr, N, BN_ROWS)
    ea4p = _encode_pack(edge_attr.reshape(E // 2, 2 * de), wd, bd,
                        E // 2, 1000)
    ea4f = ea4p.reshape(NCHUNK * (E // 2), CH)

    layers = [
        (L0_W1, L0_b1, L0_g, L0_bb, L0_W2, L0_b2),
        (L1_W1, L1_b1, L1_g, L1_bb, L1_W2, L1_b2),
        (L2_W1, L2_b1, L2_g, L2_bb, L2_W2, L2_b2),
    ]
    for (W1, b1, g, bb, W2, b2) in layers:
        aggf = _sc_aggregate(h4, ea4f, src, dst)
        agg4 = aggf.reshape(NCHUNK, N, CH)
        w1r = W1.reshape(NCHUNK, CH, H)
        w2r = W2.reshape(H, NCHUNK, CH).transpose(1, 0, 2)
        z1, stats = _mlp1(h4, agg4, w1r, b1.reshape(1, H))
        h4 = _mlp2(z1, stats, g.reshape(1, H), bb.reshape(1, H),
                   w2r, b2.reshape(NCHUNK, 1, CH))

    batch3 = batch.reshape(N // BN_ROWS, 1, BN_ROWS)
    return _pool(h4, batch3, Wout, bout.reshape(1, NCLS))


# speedup vs baseline: 1.9253x; 1.4100x over previous
import functools

import jax
import jax.numpy as jnp
from jax import lax
from jax.experimental import pallas as pl
from jax.experimental.pallas import tpu as pltpu
from jax.experimental.pallas import tpu_sc as plsc

N = 10000
E = 160000
D = 256
H = 512
G = 128
NCLS = 10

CH = 128
NCHUNK = H // CH

NSUB = 16
K = 40
EPS = E // NSUB
NBLK = EPS // K
NPAD = 10240
ROWS = NPAD // NSUB
ZR = 40
BN_ROWS = 1000
F32 = jnp.float32


def _sc_aggregate(h4, ea4f, src, dst):
    mesh = plsc.VectorSubcoreMesh(core_axis_name="c", subcore_axis_name="s")
    vg = CH // 16

    @functools.partial(
        pl.kernel,
        mesh=mesh,
        out_type=jax.ShapeDtypeStruct((NCHUNK * N, CH), F32),
        scratch_types=[
            pltpu.VMEM_SHARED((NPAD, CH), F32),
            pltpu.VMEM((K,), jnp.int32),
            pltpu.VMEM((K,), jnp.int32),
            pltpu.VMEM((K,), jnp.int32),
            pltpu.VMEM((K,), jnp.int32),
            pltpu.VMEM((K,), jnp.int32),
            pltpu.VMEM((K,), jnp.int32),
            pltpu.VMEM((K,), jnp.int32),
            pltpu.VMEM((K,), jnp.int32),
            pltpu.VMEM((K, CH), F32),
            pltpu.VMEM((K, CH), F32),
            pltpu.VMEM((K, CH), F32),
            pltpu.VMEM((K, CH), F32),
            pltpu.VMEM((K, CH), F32),
            pltpu.SemaphoreType.DMA,
            pltpu.SemaphoreType.DMA,
            pltpu.SemaphoreType.DMA,
            pltpu.SemaphoreType.DMA,
            pltpu.SemaphoreType.DMA,
            pltpu.SemaphoreType.DMA,
            pltpu.SemaphoreType.DMA,
            pltpu.SemaphoreType.DMA,
            pltpu.SemaphoreType.DMA,
            pltpu.SemaphoreType.DMA,
            pltpu.SemaphoreType.DMA,
            pltpu.SemaphoreType.DMA,
            pltpu.SemaphoreType.DMA,
            pltpu.SemaphoreType.DMA,
        ],
    )
    def agg_kernel(h_hbm, ea_hbm, src_hbm, dst_hbm, out_hbm,
                   table, s0, s1, s2, s3, d0, d1, d2, d3,
                   h0, h1, h2, e0, e1,
                   sg0, sg1, sg2, se0, se1, semsc,
                   sr0, sr1, sr2, sr3, sd0, sd1, sd2, sd3):
        core = lax.axis_index("c")
        sub = lax.axis_index("s")
        hrows = (h0, h1, h2)
        earows = (e0, e1)
        srcv = (s0, s1, s2, s3)
        dstv = (d0, d1, d2, d3)
        semg = (sg0, sg1, sg2)
        seme = (se0, se1)
        semr = (sr0, sr1, sr2, sr3)
        semd = (sd0, sd1, sd2, sd3)

        zeros16 = jnp.zeros((16,), F32)
        row0 = sub * ROWS

        for cc in range(NCHUNK // 2):
            c = core * (NCHUNK // 2) + cc
            ebase = c * E + sub * EPS
            obase = c * N
            hview = h_hbm.at[c]

            def zb(r, cy):
                for j in range(vg):
                    h0[r, pl.ds(j * 16, 16)] = zeros16
                return cy

            lax.fori_loop(0, ZR, zb, 0)
            for z in range(ROWS // ZR):
                pltpu.sync_copy(h0, table.at[pl.ds(row0 + z * ZR, ZR)])
            plsc.subcore_barrier()

            def issue_idx(b, t):
                pi = t % 4
                pltpu.async_copy(src_hbm.at[pl.ds(sub * EPS + b * K, K)],
                                 srcv[pi], semr[pi])
                pltpu.async_copy(dst_hbm.at[pl.ds(sub * EPS + b * K, K)],
                                 dstv[pi], semd[pi])

            def issue_rows(b, t):
                pg, pe, pi = t % 3, t % 2, t % 4
                pltpu.make_async_copy(
                    src_hbm.at[pl.ds(sub * EPS + b * K, K)], srcv[pi],
                    semr[pi]).wait()
                pltpu.async_copy(hview.at[srcv[pi]], hrows[pg], semg[pg])
                pltpu.async_copy(ea_hbm.at[pl.ds(ebase + b * K, K)],
                                 earows[pe], seme[pe])

            def drain_scatter(pg, pi):
                pltpu.make_async_copy(hrows[pg], table.at[dstv[pi]],
                                      semsc).wait()

            def step(b, t):
                pg, pe, pi = t % 3, t % 2, t % 4

                @pl.when(b + 2 < NBLK)
                def _():
                    issue_idx(b + 2, t + 2)

                @pl.when(b + 1 < NBLK)
                def _():
                    issue_rows(b + 1, t + 1)

                @pl.when(b < NBLK)
                def _():
                    pltpu.make_async_copy(
                        hview.at[srcv[pi]], hrows[pg], semg[pg]).wait()
                    pltpu.make_async_copy(
                        ea_hbm.at[pl.ds(ebase + b * K, K)], earows[pe],
                        seme[pe]).wait()

                    def rel(r, cy):
                        for j in range(vg):
                            sl = pl.ds(j * 16, 16)
                            v = hrows[pg][r, sl] + earows[pe][r, sl]
                            hrows[pg][r, sl] = jnp.maximum(v, 0.0)
                        return cy

                    lax.fori_loop(0, K, rel, 0)
                    pltpu.make_async_copy(
                        dst_hbm.at[pl.ds(sub * EPS + b * K, K)], dstv[pi],
                        semd[pi]).wait()

                    @pl.when(b >= 1)
                    def _():
                        drain_scatter((t + 2) % 3, (t + 3) % 4)

                    pltpu.async_copy(hrows[pg], table.at[dstv[pi]],
                                     semsc, add=True)

            issue_idx(0, 0)
            issue_idx(1, 1)
            issue_rows(0, 0)

            def grp(g, cy):
                for t in range(12):
                    step(12 * g + t, t)
                return cy

            lax.fori_loop(0, (NBLK + 12 - 1) // 12, grp, 0)
            drain_scatter((NBLK - 1) % 3, (NBLK - 1) % 4)
            plsc.subcore_barrier()

            for z in range(ROWS // ZR):
                off = row0 + z * ZR

                @pl.when(off + ZR <= N)
                def _():
                    pltpu.sync_copy(table.at[pl.ds(off, ZR)], h0)
                    pltpu.sync_copy(h0, out_hbm.at[pl.ds(obase + off, ZR)])

            plsc.subcore_barrier()

    return agg_kernel(h4, ea4f, src, dst)


def _encode_body(x_ref, w_ref, b_ref, o_ref):
    xb = x_ref[...]
    for c in range(NCHUNK):
        z = jnp.dot(xb, w_ref[c], preferred_element_type=F32) + b_ref[c]
        o_ref[c] = z.astype(o_ref.dtype)


def _encode(x, wr, br, rows, block_rows, dtype=F32):
    din = x.shape[1]
    grid = (rows // block_rows,)
    return pl.pallas_call(
        _encode_body,
        grid=grid,
        in_specs=[
            pl.BlockSpec((block_rows, din), lambda i: (i, 0)),
            pl.BlockSpec((NCHUNK, din, CH), lambda i: (0, 0, 0)),
            pl.BlockSpec((NCHUNK, 1, CH), lambda i: (0, 0, 0)),
        ],
        out_specs=pl.BlockSpec((NCHUNK, block_rows, CH), lambda i: (0, i, 0)),
        out_shape=jax.ShapeDtypeStruct((NCHUNK, rows, CH), dtype),
    )(x, wr, br)


def _mlp1_body(h_ref, a_ref, w_ref, b_ref, z_ref, s_ref):
    acc = jnp.zeros((BN_ROWS, H), F32)
    for c in range(NCHUNK):
        acc += jnp.dot(h_ref[c] + a_ref[c], w_ref[c],
                       preferred_element_type=F32)
    z = acc + b_ref[...]
    z_ref[...] = z
    delta = jnp.concatenate(
        [jnp.sum(z, axis=0, keepdims=True),
         jnp.sum(z * z, axis=0, keepdims=True)], axis=0)

    @pl.when(pl.program_id(0) == 0)
    def _():
        s_ref[...] = delta

    @pl.when(pl.program_id(0) > 0)
    def _():
        s_ref[...] += delta


def _mlp1(h4, agg4, w1r, b1r):
    grid = (N // BN_ROWS,)
    return pl.pallas_call(
        _mlp1_body,
        grid=grid,
        in_specs=[
            pl.BlockSpec((NCHUNK, BN_ROWS, CH), lambda i: (0, i, 0)),
            pl.BlockSpec((NCHUNK, BN_ROWS, CH), lambda i: (0, i, 0)),
            pl.BlockSpec((NCHUNK, CH, H), lambda i: (0, 0, 0)),
            pl.BlockSpec((1, H), lambda i: (0, 0)),
        ],
        out_specs=[
            pl.BlockSpec((BN_ROWS, H), lambda i: (i, 0)),
            pl.BlockSpec((2, H), lambda i: (0, 0)),
        ],
        out_shape=[
            jax.ShapeDtypeStruct((N, H), F32),
            jax.ShapeDtypeStruct((2, H), F32),
        ],
    )(h4, agg4, w1r, b1r)


def _mlp2_body(z_ref, s_ref, g_ref, bb_ref, w_ref, b_ref, o_ref):
    mu = s_ref[0:1, :] * (1.0 / N)
    var = s_ref[1:2, :] * (1.0 / N) - mu * mu
    scale = g_ref[...] * lax.rsqrt(var + 1e-5)
    shift = bb_ref[...] - mu * scale
    zn = jnp.maximum(z_ref[...] * scale + shift, 0.0)
    for c in range(NCHUNK):
        o_ref[c] = jnp.maximum(
            jnp.dot(zn, w_ref[c], preferred_element_type=F32) + b_ref[c], 0.0)


def _mlp2(z1, stats, gr, bbr, w2r, b2r):
    grid = (N // BN_ROWS,)
    return pl.pallas_call(
        _mlp2_body,
        grid=grid,
        in_specs=[
            pl.BlockSpec((BN_ROWS, H), lambda i: (i, 0)),
            pl.BlockSpec((2, H), lambda i: (0, 0)),
            pl.BlockSpec((1, H), lambda i: (0, 0)),
            pl.BlockSpec((1, H), lambda i: (0, 0)),
            pl.BlockSpec((NCHUNK, H, CH), lambda i: (0, 0, 0)),
            pl.BlockSpec((NCHUNK, 1, CH), lambda i: (0, 0, 0)),
        ],
        out_specs=pl.BlockSpec((NCHUNK, BN_ROWS, CH), lambda i: (0, i, 0)),
        out_shape=jax.ShapeDtypeStruct((NCHUNK, N, CH), F32),
    )(z1, stats, gr, bbr, w2r, b2r)


def _pool_body(h_ref, batch_ref, wo_ref, bo_ref, o_ref):
    b = batch_ref[0, 0, :]
    oh = (b[:, None] == lax.broadcasted_iota(jnp.int32, (BN_ROWS, G), 1)
          ).astype(F32)
    hb = jnp.concatenate([h_ref[c] for c in range(NCHUNK)], axis=1)
    pooled = lax.dot_general(oh, hb, (((0,), (0,)), ((), ())),
                             preferred_element_type=F32)
    contrib = jnp.dot(pooled, wo_ref[...], preferred_element_type=F32)

    @pl.when(pl.program_id(0) == 0)
    def _():
        o_ref[...] = contrib + bo_ref[...]

    @pl.when(pl.program_id(0) > 0)
    def _():
        o_ref[...] += contrib


def _pool(h4, batch3, wout, bout2):
    grid = (N // BN_ROWS,)
    nb = N // BN_ROWS
    return pl.pallas_call(
        _pool_body,
        grid=grid,
        in_specs=[
            pl.BlockSpec((NCHUNK, BN_ROWS, CH), lambda i: (0, i, 0)),
            pl.BlockSpec((1, 1, BN_ROWS), lambda i: (i, 0, 0)),
            pl.BlockSpec((H, NCLS), lambda i: (0, 0)),
            pl.BlockSpec((1, NCLS), lambda i: (0, 0)),
        ],
        out_specs=pl.BlockSpec((G, NCLS), lambda i: (0, 0)),
        out_shape=jax.ShapeDtypeStruct((G, NCLS), F32),
    )(h4, batch3, wout, bout2)


def kernel(x, edge_index, batch, edge_attr, Wn, bnv, We, bev,
           L0_W1, L0_b1, L0_g, L0_bb, L0_W2, L0_b2,
           L1_W1, L1_b1, L1_g, L1_bb, L1_W2, L1_b2,
           L2_W1, L2_b1, L2_g, L2_bb, L2_W2, L2_b2,
           Wout, bout):
    src = edge_index[0]
    dst = edge_index[1]

    wnr = Wn.reshape(D, NCHUNK, CH).transpose(1, 0, 2)
    bnr = bnv.reshape(NCHUNK, 1, CH)

    wer = We.reshape(edge_attr.shape[1], NCHUNK, CH).transpose(1, 0, 2)
    ber = bev.reshape(NCHUNK, 1, CH)

    h4 = _encode(x, wnr, bnr, N, BN_ROWS)
    ea4 = _encode(edge_attr, wer, ber, E, 2000)
    ea4f = ea4.reshape(NCHUNK * E, CH)

    layers = [
        (L0_W1, L0_b1, L0_g, L0_bb, L0_W2, L0_b2),
        (L1_W1, L1_b1, L1_g, L1_bb, L1_W2, L1_b2),
        (L2_W1, L2_b1, L2_g, L2_bb, L2_W2, L2_b2),
    ]
    for (W1, b1, g, bb, W2, b2) in layers:
        aggf = _sc_aggregate(h4, ea4f, src, dst)
        agg4 = aggf.reshape(NCHUNK, N, CH)
        w1r = W1.reshape(NCHUNK, CH, H)
        w2r = W2.reshape(H, NCHUNK, CH).transpose(1, 0, 2)
        z1, stats = _mlp1(h4, agg4, w1r, b1.reshape(1, H))
        h4 = _mlp2(z1, stats, g.reshape(1, H), bb.reshape(1, H),
                   w2r, b2.reshape(NCHUNK, 1, CH))

    batch3 = batch.reshape(N // BN_ROWS, 1, BN_ROWS)
    return _pool(h4, batch3, Wout, bout.reshape(1, NCLS))
